# baseline (speedup 1.0000x reference)
import functools

import jax
import jax.numpy as jnp
from jax import lax
from jax.experimental import pallas as pl
from jax.experimental.pallas import tpu as pltpu
from jax.experimental.pallas import tpu_sc as plsc

B, D_IN, D_H, D_OUT = 64, 2048, 2048, 2048
K_MEM, D_RET = 100, 5
N1 = 2
N2 = 2
K1 = D_IN // N1
K2 = D_H // N2


def _body(x_ref, tgt_ref, w1_ref, b1_ref, lnw_ref, lnb_ref, w2_ref, b2_ref,
          out_ref, scal_ref, dfn_ref, featsn_ref, feats_out_ref,
          acc_ref, feats_ref):
    i = pl.program_id(0)

    @pl.when(i == 0)
    def _init():
        acc_ref[...] = jnp.zeros_like(acc_ref)

    @pl.when(i < N1)
    def _mm1():
        acc_ref[...] += jnp.dot(x_ref[...], w1_ref[...],
                                preferred_element_type=jnp.float32)

    @pl.when(i == N1 - 1)
    def _ln():
        h = acc_ref[...] + b1_ref[...]
        mu = jnp.mean(h, axis=-1, keepdims=True)
        var = jnp.mean((h - mu) ** 2, axis=-1, keepdims=True)
        ln = (h - mu) / jnp.sqrt(var + 1e-5) * lnw_ref[...] + lnb_ref[...]
        feats = jnp.maximum(ln, 0.0)
        for j in range(N2):
            feats_ref[j] = feats[:, j * K2:(j + 1) * K2]
        feats_out_ref[...] = feats
        acc_ref[...] = jnp.zeros_like(acc_ref)

    @pl.when(i >= N1)
    def _mm2():
        j = i - N1
        acc_ref[...] += jnp.dot(feats_ref[j], w2_ref[...],
                                preferred_element_type=jnp.float32)

    @pl.when(i == N1)
    def _retrieve_inputs():
        feats = jnp.concatenate([feats_ref[j] for j in range(N2)], axis=1)
        keys = jnp.mean(feats, axis=0, keepdims=True)
        keys_n = jnp.maximum(jnp.sqrt(jnp.sum(keys * keys)), 1e-8)
        rn = jnp.sqrt(jnp.sum(feats * feats, axis=1, keepdims=True))
        dots = jnp.sum(feats * keys, axis=1, keepdims=True)
        distances = dots / (jnp.maximum(rn, 1e-8) * keys_n)

        feats_n = jnp.mean(feats / jnp.maximum(rn, 1e-12), axis=0,
                           keepdims=True)
        fn_n = jnp.maximum(jnp.sqrt(jnp.sum(feats_n * feats_n)), 1e-8)
        featsn_ref[...] = feats_n

        lane = jax.lax.broadcasted_iota(jnp.int32, (B, 128), 1)
        dfn_ref[...] = jnp.where(lane == 0, distances,
                                 jnp.where(lane == 1, fn_n, 0.0))

    @pl.when(i == N1 + N2 - 1)
    def _final():
        out = acc_ref[...] + b2_ref[...]
        out_ref[...] = out
        t = tgt_ref[...]
        d = out - t
        sq_mean = jnp.mean(d * d)
        rmse = jnp.sqrt(sq_mean)
        nmse = sq_mean / jnp.mean(t * t)
        loss = rmse + nmse
        lane = jax.lax.broadcasted_iota(jnp.int32, (1, 128), 1)
        scal_ref[...] = jnp.where(lane == 0, loss, 0.0)


def _i16():
    return lax.iota(jnp.int32, 16)


def _perm(x, p):
    dnums = lax.GatherDimensionNumbers(offset_dims=(),
                                       collapsed_slice_dims=(0,),
                                       start_index_map=(0,))
    return lax.gather(x, p[:, None], dnums, (1,),
                      mode=lax.GatherScatterMode.PROMISE_IN_BOUNDS)


def _splat_min(x):
    for s in (8, 4, 2, 1):
        x = jnp.minimum(x, _perm(x, _i16() ^ s))
    return x


def _splat_sum(x):
    for s in (8, 4, 2, 1):
        x = x + _perm(x, _i16() ^ s)
    return x


def _sqrt_newton(x):
    xb = lax.bitcast_convert_type(x, jnp.int32)
    y = lax.bitcast_convert_type(jnp.int32(0x5F3759DF) - (xb >> 1),
                                 jnp.float32)
    for _ in range(3):
        y = y * (1.5 - 0.5 * x * y * y)
    return jnp.where(x < 1e-30, 0.0, x * y)


@functools.partial(
    pl.kernel,
    mesh=plsc.VectorSubcoreMesh(core_axis_name="c", subcore_axis_name="s"),
    out_type=jax.ShapeDtypeStruct((16,), jnp.float32),
    scratch_types=[
        pltpu.VMEM((B,), jnp.float32),
        pltpu.VMEM((16,), jnp.float32),
        pltpu.VMEM((D_H,), jnp.float32),
        pltpu.VMEM((16, 16, 128), jnp.float32),
        pltpu.VMEM((16,), jnp.float32),
        pltpu.SemaphoreType.DMA,
    ],
)
def _sc_retrieve(dists_hbm, fn16_hbm, featsn_hbm, feats_hbm, out_hbm,
                 dv, fnsc, fnv, rows_v, resv, sem):
    cid = lax.axis_index("c")
    sid = lax.axis_index("s")

    @pl.when((cid == 0) & (sid == 0))
    def _():
        pltpu.sync_copy(dists_hbm, dv)
        pltpu.sync_copy(fn16_hbm, fnsc)
        pltpu.sync_copy(featsn_hbm, fnv)

        i16 = _i16()
        regs = [dv[pl.ds(16 * v, 16)] for v in range(4)]
        idxs = [i16 + 16 * v for v in range(4)]

        idx_sel = jnp.zeros((16,), jnp.int32)
        for k in range(D_RET):
            m = _splat_min(jnp.minimum(jnp.minimum(regs[0], regs[1]),
                                       jnp.minimum(regs[2], regs[3])))
            first = jnp.full((16,), 4 * B, jnp.int32)
            for v in range(4):
                cand = _splat_min(jnp.where(regs[v] == m, idxs[v], 4 * B))
                first = jnp.minimum(first, cand)
            idx_sel = jnp.where(i16 == k, first, idx_sel)
            regs = [jnp.where(idxs[v] == first, jnp.float32(99.0), regs[v])
                    for v in range(4)]

        pltpu.async_copy(feats_hbm.at[idx_sel], rows_v, sem).wait()

        dot = jnp.zeros((16,), jnp.float32)
        s2 = jnp.zeros((16,), jnp.float32)
        for p in range(16):
            for qq in range(8):
                sl = pl.ds(qq * 16, 16)
                acc = (rows_v[0, p, sl] + rows_v[1, p, sl]
                       + rows_v[2, p, sl] + rows_v[3, p, sl]
                       + rows_v[4, p, sl])
                dot = dot + fnv[pl.ds(p * 128 + qq * 16, 16)] * acc
                s2 = s2 + acc * acc
        dot = _splat_sum(dot)
        s2 = _splat_sum(s2)

        fn_n = fnsc[...]

        sm_norm = _sqrt_newton(s2) / D_RET
        sm_n12 = jnp.maximum(sm_norm, 1e-12)
        c_n = jnp.maximum(sm_norm / sm_n12, 1e-8)
        cos = (dot / D_RET / sm_n12) / (fn_n * c_n)
        dist = 1.0 - cos
        adjusted_lr = 2e-05 * jnp.exp(-dist * 0.01)

        resv[...] = jnp.where(i16 == 0, dist,
                              jnp.where(i16 == 1, adjusted_lr, 0.0))
        pltpu.sync_copy(resv, out_hbm)


def kernel(inputs, target, W1, b1, ln_w, ln_b, W2, b2):
    grid = (N1 + N2,)
    out, scal, dfn, featsn, feats = pl.pallas_call(
        _body,
        grid=grid,
        in_specs=[
            pl.BlockSpec((B, K1), lambda i: (0, jnp.minimum(i, N1 - 1))),
            pl.BlockSpec((B, D_OUT), lambda i: (0, 0)),
            pl.BlockSpec((K1, D_H), lambda i: (jnp.minimum(i, N1 - 1), 0)),
            pl.BlockSpec((1, D_H), lambda i: (0, 0)),
            pl.BlockSpec((1, D_H), lambda i: (0, 0)),
            pl.BlockSpec((1, D_H), lambda i: (0, 0)),
            pl.BlockSpec((K2, D_OUT), lambda i: (jnp.maximum(i - N1, 0), 0)),
            pl.BlockSpec((1, D_OUT), lambda i: (0, 0)),
        ],
        out_specs=[
            pl.BlockSpec((B, D_OUT), lambda i: (0, 0)),
            pl.BlockSpec((1, 128), lambda i: (0, 0)),
            pl.BlockSpec((B, 128), lambda i: (0, 0)),
            pl.BlockSpec((1, D_H), lambda i: (0, 0)),
            pl.BlockSpec((B, D_H), lambda i: (0, 0)),
        ],
        out_shape=[
            jax.ShapeDtypeStruct((B, D_OUT), jnp.float32),
            jax.ShapeDtypeStruct((1, 128), jnp.float32),
            jax.ShapeDtypeStruct((B, 128), jnp.float32),
            jax.ShapeDtypeStruct((1, D_H), jnp.float32),
            jax.ShapeDtypeStruct((B, D_H), jnp.float32),
        ],
        scratch_shapes=[
            pltpu.VMEM((B, D_H), jnp.float32),
            pltpu.VMEM((N2, B, K2), jnp.float32),
        ],
        compiler_params=pltpu.CompilerParams(
            dimension_semantics=("arbitrary",),
        ),
    )(
        inputs, target, W1,
        b1.reshape(1, D_H), ln_w.reshape(1, D_H), ln_b.reshape(1, D_H),
        W2, b2.reshape(1, D_OUT),
    )
    sc = _sc_retrieve(dfn[:, 0], jnp.full((16,), dfn[0, 1], jnp.float32),
                      featsn.reshape(D_H), feats.reshape(B, 16, 128))
    return (out, scal[0, 0], sc[0], sc[1])


# speedup vs baseline: 2.5567x; 2.5567x over previous
import jax
import jax.numpy as jnp
from jax.experimental import pallas as pl
from jax.experimental.pallas import tpu as pltpu

B, D_IN, D_H, D_OUT = 64, 2048, 2048, 2048
K_MEM, D_RET = 100, 5
N1 = 2
N2 = 2
K1 = D_IN // N1
K2 = D_H // N2


def _body(x_ref, tgt_ref, w1a_ref, w1b_ref, b1_ref, lnw_ref, lnb_ref,
          w2a_ref, w2b_ref, b2_ref,
          out_ref, scal_ref, acc_ref, feats_ref):
    i = pl.program_id(0)
    H2 = D_H // 2

    @pl.when(i == 0)
    def _init():
        acc_ref[...] = jnp.zeros_like(acc_ref)

    @pl.when(i < N1)
    def _mm1():
        x = x_ref[...]
        acc_ref[:, :H2] += jnp.dot(x, w1a_ref[...],
                                   preferred_element_type=jnp.float32)
        acc_ref[:, H2:] += jnp.dot(x, w1b_ref[...],
                                   preferred_element_type=jnp.float32)

    @pl.when(i == N1 - 1)
    def _ln():
        h = acc_ref[...] + b1_ref[...]
        mu = jnp.mean(h, axis=-1, keepdims=True)
        var = jnp.mean((h - mu) ** 2, axis=-1, keepdims=True)
        ln = (h - mu) / jnp.sqrt(var + 1e-5) * lnw_ref[...] + lnb_ref[...]
        feats = jnp.maximum(ln, 0.0)
        for j in range(N2):
            feats_ref[j] = feats[:, j * K2:(j + 1) * K2]
        acc_ref[...] = jnp.zeros_like(acc_ref)

    @pl.when(i >= N1)
    def _mm2():
        j = i - N1
        f = feats_ref[j]
        acc_ref[:, :H2] += jnp.dot(f, w2a_ref[...],
                                   preferred_element_type=jnp.float32)
        acc_ref[:, H2:] += jnp.dot(f, w2b_ref[...],
                                   preferred_element_type=jnp.float32)

    @pl.when(i == N1)
    def _retrieve():
        feats = jnp.concatenate([feats_ref[j] for j in range(N2)], axis=1)
        keys = jnp.mean(feats, axis=0, keepdims=True)
        keys_n = jnp.maximum(jnp.sqrt(jnp.sum(keys * keys)), 1e-8)
        rn = jnp.sqrt(jnp.sum(feats * feats, axis=1, keepdims=True))
        dots = jnp.sum(feats * keys, axis=1, keepdims=True)
        distances = dots / (jnp.maximum(rn, 1e-8) * keys_n)

        iota = jax.lax.broadcasted_iota(jnp.int32, (B, 1), 0)
        work = distances
        sel = jnp.zeros((B, 1), dtype=jnp.float32)
        for _ in range(D_RET):
            m = jnp.min(work)
            first = jnp.min(jnp.where(work == m, iota, B))
            pick = iota == first
            sel = jnp.where(pick, 1.0, sel)
            work = jnp.where(pick, 99.0, work)

        support_mean = jnp.sum(feats * sel, axis=0, keepdims=True) / D_RET
        sm_n = jnp.maximum(jnp.sqrt(jnp.sum(support_mean * support_mean)),
                           1e-12)
        centers = support_mean / sm_n
        feats_n = jnp.mean(feats / jnp.maximum(rn, 1e-12), axis=0,
                           keepdims=True)
        fn_n = jnp.maximum(jnp.sqrt(jnp.sum(feats_n * feats_n)), 1e-8)
        c_n = jnp.maximum(jnp.sqrt(jnp.sum(centers * centers)), 1e-8)
        cos = jnp.sum(feats_n * centers) / (fn_n * c_n)
        dist = 1.0 - cos
        adjusted_lr = 2e-05 * jnp.exp(-dist * 0.01)

        lane = jax.lax.broadcasted_iota(jnp.int32, (1, 128), 1)
        scal_ref[...] = jnp.where(lane == 1, dist,
                                  jnp.where(lane == 2, adjusted_lr, 0.0))

    @pl.when(i == N1 + N2 - 1)
    def _final():
        out = acc_ref[...] + b2_ref[...]
        out_ref[...] = out
        t = tgt_ref[...]
        d = out - t
        sq_mean = jnp.mean(d * d)
        rmse = jnp.sqrt(sq_mean)
        nmse = sq_mean / jnp.mean(t * t)
        loss = rmse + nmse
        lane = jax.lax.broadcasted_iota(jnp.int32, (1, 128), 1)
        scal_ref[...] = jnp.where(lane == 0, loss, scal_ref[...])


def kernel(inputs, target, W1, b1, ln_w, ln_b, W2, b2):
    grid = (N1 + N2,)
    out, scal = pl.pallas_call(
        _body,
        grid=grid,
        in_specs=[
            pl.BlockSpec((B, K1), lambda i: (0, jnp.minimum(i, N1 - 1))),
            pl.BlockSpec((B, D_OUT), lambda i: (0, 0)),
            pl.BlockSpec((K1, D_H // 2),
                         lambda i: (jnp.minimum(i, N1 - 1), 0)),
            pl.BlockSpec((K1, D_H // 2),
                         lambda i: (jnp.minimum(i, N1 - 1), 1)),
            pl.BlockSpec((1, D_H), lambda i: (0, 0)),
            pl.BlockSpec((1, D_H), lambda i: (0, 0)),
            pl.BlockSpec((1, D_H), lambda i: (0, 0)),
            pl.BlockSpec((K2, D_OUT // 2),
                         lambda i: (jnp.maximum(i - N1, 0), 0)),
            pl.BlockSpec((K2, D_OUT // 2),
                         lambda i: (jnp.maximum(i - N1, 0), 1)),
            pl.BlockSpec((1, D_OUT), lambda i: (0, 0)),
        ],
        out_specs=[
            pl.BlockSpec((B, D_OUT), lambda i: (0, 0)),
            pl.BlockSpec((1, 128), lambda i: (0, 0)),
        ],
        out_shape=[
            jax.ShapeDtypeStruct((B, D_OUT), jnp.float32),
            jax.ShapeDtypeStruct((1, 128), jnp.float32),
        ],
        scratch_shapes=[
            pltpu.VMEM((B, D_H), jnp.float32),
            pltpu.VMEM((N2, B, K2), jnp.float32),
        ],
        compiler_params=pltpu.CompilerParams(
            dimension_semantics=("arbitrary",),
        ),
    )(
        inputs, target, W1, W1,
        b1.reshape(1, D_H), ln_w.reshape(1, D_H), ln_b.reshape(1, D_H),
        W2, W2, b2.reshape(1, D_OUT),
    )
    return (out, scal[0, 0], scal[0, 1], scal[0, 2])


# speedup vs baseline: 2.7066x; 1.0586x over previous
import jax
import jax.numpy as jnp
from jax.experimental import pallas as pl
from jax.experimental.pallas import tpu as pltpu

B, D_IN, D_H, D_OUT = 64, 2048, 2048, 2048
K_MEM, D_RET = 100, 5
N1 = 2
N2 = 2
K1 = D_IN // N1
K2 = D_H // N2


def _body(x_ref, tgt_ref, w1_ref, b1_ref, lnw_ref, lnb_ref, w2_ref, b2_ref,
          out_ref, scal_ref, acc_ref, feats_ref):
    i = pl.program_id(0)

    @pl.when(i == 0)
    def _init():
        acc_ref[...] = jnp.zeros_like(acc_ref)

    @pl.when(i < N1)
    def _mm1():
        acc_ref[...] += jnp.dot(x_ref[...], w1_ref[...],
                                preferred_element_type=jnp.float32)

    @pl.when(i == N1 - 1)
    def _ln():
        h = acc_ref[...] + b1_ref[...]
        mu = jnp.mean(h, axis=-1, keepdims=True)
        var = jnp.mean((h - mu) ** 2, axis=-1, keepdims=True)
        ln = (h - mu) / jnp.sqrt(var + 1e-5) * lnw_ref[...] + lnb_ref[...]
        feats = jnp.maximum(ln, 0.0)
        for j in range(N2):
            feats_ref[j] = feats[:, j * K2:(j + 1) * K2]
        acc_ref[...] = jnp.zeros_like(acc_ref)

    @pl.when(i >= N1)
    def _mm2():
        j = i - N1
        acc_ref[...] += jnp.dot(feats_ref[j], w2_ref[...],
                                preferred_element_type=jnp.float32)

    @pl.when(i == N1)
    def _retrieve():
        feats = jnp.concatenate([feats_ref[j] for j in range(N2)], axis=1)
        keys = jnp.mean(feats, axis=0, keepdims=True)
        keys_n = jnp.maximum(jnp.sqrt(jnp.sum(keys * keys)), 1e-8)
        rn = jnp.sqrt(jnp.sum(feats * feats, axis=1, keepdims=True))
        dots = jnp.sum(feats * keys, axis=1, keepdims=True)
        distances = dots / (jnp.maximum(rn, 1e-8) * keys_n)

        iota = jax.lax.broadcasted_iota(jnp.int32, (B, 1), 0)
        work = distances
        sel = jnp.zeros((B, 1), dtype=jnp.float32)
        for _ in range(D_RET):
            m = jnp.min(work)
            first = jnp.min(jnp.where(work == m, iota, B))
            pick = iota == first
            sel = jnp.where(pick, 1.0, sel)
            work = jnp.where(pick, 99.0, work)

        support_mean = jnp.sum(feats * sel, axis=0, keepdims=True) / D_RET
        sm_n = jnp.maximum(jnp.sqrt(jnp.sum(support_mean * support_mean)),
                           1e-12)
        centers = support_mean / sm_n
        feats_n = jnp.mean(feats / jnp.maximum(rn, 1e-12), axis=0,
                           keepdims=True)
        fn_n = jnp.maximum(jnp.sqrt(jnp.sum(feats_n * feats_n)), 1e-8)
        c_n = jnp.maximum(jnp.sqrt(jnp.sum(centers * centers)), 1e-8)
        cos = jnp.sum(feats_n * centers) / (fn_n * c_n)
        dist = 1.0 - cos
        adjusted_lr = 2e-05 * jnp.exp(-dist * 0.01)

        lane = jax.lax.broadcasted_iota(jnp.int32, (1, 128), 1)
        scal_ref[...] = jnp.where(lane == 1, dist,
                                  jnp.where(lane == 2, adjusted_lr, 0.0))

    @pl.when(i == N1 + N2 - 1)
    def _final():
        out = acc_ref[...] + b2_ref[...]
        out_ref[...] = out
        t = tgt_ref[...]
        d = out - t
        sq_mean = jnp.mean(d * d)
        rmse = jnp.sqrt(sq_mean)
        nmse = sq_mean / jnp.mean(t * t)
        loss = rmse + nmse
        lane = jax.lax.broadcasted_iota(jnp.int32, (1, 128), 1)
        scal_ref[...] = jnp.where(lane == 0, loss, scal_ref[...])


def kernel(inputs, target, W1, b1, ln_w, ln_b, W2, b2):
    grid = (N1 + N2,)
    out, scal = pl.pallas_call(
        _body,
        grid=grid,
        in_specs=[
            pl.BlockSpec((B, K1), lambda i: (0, jnp.minimum(i, N1 - 1))),
            pl.BlockSpec((B, D_OUT), lambda i: (0, 0)),
            pl.BlockSpec((K1, D_H), lambda i: (jnp.minimum(i, N1 - 1), 0)),
            pl.BlockSpec((1, D_H), lambda i: (0, 0)),
            pl.BlockSpec((1, D_H), lambda i: (0, 0)),
            pl.BlockSpec((1, D_H), lambda i: (0, 0)),
            pl.BlockSpec((K2, D_OUT), lambda i: (jnp.maximum(i - N1, 0), 0)),
            pl.BlockSpec((1, D_OUT), lambda i: (0, 0)),
        ],
        out_specs=[
            pl.BlockSpec((B, D_OUT), lambda i: (0, 0)),
            pl.BlockSpec((1, 128), lambda i: (0, 0)),
        ],
        out_shape=[
            jax.ShapeDtypeStruct((B, D_OUT), jnp.float32),
            jax.ShapeDtypeStruct((1, 128), jnp.float32),
        ],
        scratch_shapes=[
            pltpu.VMEM((B, D_H), jnp.float32),
            pltpu.VMEM((N2, B, K2), jnp.float32),
        ],
        compiler_params=pltpu.CompilerParams(
            dimension_semantics=("arbitrary",),
        ),
    )(
        inputs, target, W1,
        b1.reshape(1, D_H), ln_w.reshape(1, D_H), ln_b.reshape(1, D_H),
        W2, b2.reshape(1, D_OUT),
    )
    return (out, scal[0, 0], scal[0, 1], scal[0, 2])
